# Initial kernel scaffold; baseline (speedup 1.0000x reference)
import jax
import jax.numpy as jnp
from jax.experimental import pallas as pl


def kernel(x, edge_index, edge_attr, batch, W1, b1, W2, b2, Wc, bc):
    raise NotImplementedError("write your pallas kernel here")


# speedup vs baseline: 5.5651x; 5.5651x over previous
import functools

import jax
import jax.numpy as jnp
from jax import lax
from jax.experimental import pallas as pl
from jax.experimental.pallas import tpu as pltpu
from jax.experimental.pallas import tpu_sc as plsc

N = 50000
E = 800000
NPAD = 50176
RPT = NPAD // 16
ZB = 224
GB = 128
NGRP = E // GB


def _sc_mesh():
    return plsc.VectorSubcoreMesh(core_axis_name="c", subcore_axis_name="s")


def _zero_zbuf(zbuf, cols):
    zeros16 = jnp.zeros((16,), jnp.float32)

    def body(r, carry):
        for k in range(cols // 16):
            zbuf[r, pl.ds(k * 16, 16)] = zeros16
        return carry

    lax.fori_loop(0, ZB, body, 0)


def _zero_acc(acc, zbuf, sid):
    for k in range(RPT // ZB):
        pltpu.sync_copy(zbuf, acc.at[pl.ds((sid * (RPT // ZB) + k) * ZB, ZB)])


def _prop_fsplit():
    @functools.partial(
        pl.kernel,
        out_type=jax.ShapeDtypeStruct((4, NPAD, 32), jnp.float32),
        mesh=_sc_mesh(),
        compiler_params=pltpu.CompilerParams(use_tc_tiling_on_sc=False),
        scratch_types=[
            pltpu.VMEM_SHARED((NPAD, 32), jnp.float32),
            pltpu.VMEM((GB, 32), jnp.float32),
            pltpu.VMEM((ZB, 32), jnp.float32),
            pltpu.VMEM((GB,), jnp.int32),
            pltpu.VMEM((GB,), jnp.int32),
            pltpu.VMEM((1, GB), jnp.int32),
        ],
    )
    def prop(g_hbm, src_hbm, dst2_hbm, out_hbm,
             acc, stage, zbuf, srcv, gidx, dstv):
        cid = lax.axis_index("c")
        sid = lax.axis_index("s")
        _zero_zbuf(zbuf, 32)
        ngrp = jnp.where(sid < 10, 391, 390)
        lo = sid * 390 + jnp.minimum(sid, 10)

        for li in range(2):
            p = 2 * cid + li
            pv4 = jnp.full((16,), 4, jnp.int32)
            pvp = jnp.full((16,), p, jnp.int32)
            _zero_acc(acc, zbuf, sid)
            plsc.subcore_barrier()

            def group_body(k, carry):
                g = lo + k
                pltpu.sync_copy(src_hbm.at[pl.ds(g * GB, GB)], srcv)
                pltpu.sync_copy(dst2_hbm.at[pl.ds(g, 1)], dstv)
                for j in range(GB // 16):
                    sl = pl.ds(j * 16, 16)
                    gidx[sl] = srcv[sl] * pv4 + pvp
                pltpu.sync_copy(g_hbm.at[gidx], stage)
                pltpu.sync_copy(stage, acc.at[dstv.at[0]], add=True)
                return carry

            lax.fori_loop(0, ngrp, group_body, 0)
            plsc.subcore_barrier()
            pltpu.sync_copy(acc.at[pl.ds(sid * RPT, RPT)],
                            out_hbm.at[p, pl.ds(sid * RPT, RPT)])
            plsc.subcore_barrier()

    return prop


def _prop_esplit():
    @functools.partial(
        pl.kernel,
        out_type=jax.ShapeDtypeStruct((2, NPAD, 16), jnp.float32),
        mesh=_sc_mesh(),
        compiler_params=pltpu.CompilerParams(use_tc_tiling_on_sc=False),
        scratch_types=[
            pltpu.VMEM_SHARED((NPAD, 16), jnp.float32),
            pltpu.VMEM((GB, 16), jnp.float32),
            pltpu.VMEM((ZB, 16), jnp.float32),
            pltpu.VMEM((GB,), jnp.int32),
            pltpu.VMEM((1, GB), jnp.int32),
        ],
    )
    def prop(g_hbm, src_hbm, dst2_hbm, out_hbm,
             acc, stage, zbuf, gidx, dstv):
        cid = lax.axis_index("c")
        sid = lax.axis_index("s")
        _zero_zbuf(zbuf, 16)
        half = NGRP // 2
        ngrp = jnp.where(sid < 5, 196, 195)
        lo = cid * half + sid * 195 + jnp.minimum(sid, 5)

        _zero_acc(acc, zbuf, sid)
        plsc.subcore_barrier()

        def group_body(k, carry):
            g = lo + k
            pltpu.sync_copy(src_hbm.at[pl.ds(g * GB, GB)], gidx)
            pltpu.sync_copy(dst2_hbm.at[pl.ds(g, 1)], dstv)
            pltpu.sync_copy(g_hbm.at[gidx], stage)
            pltpu.sync_copy(stage, acc.at[dstv.at[0]], add=True)
            return carry

        lax.fori_loop(0, ngrp, group_body, 0)
        plsc.subcore_barrier()
        pltpu.sync_copy(acc.at[pl.ds(sid * RPT, RPT)],
                        out_hbm.at[cid, pl.ds(sid * RPT, RPT)])

    return prop


BM = 448
GRID = NPAD // BM


def _dis(d0b, d1b):
    return lax.rsqrt(d0b[...][:, 0:1] + d1b[...][:, 0:1] + 1.0)


def _spec(shape2d):
    return pl.BlockSpec((BM, shape2d), lambda i: (i, 0))


def _mm_scale(xp, d0, d1, w):
    kdim, fout = xp.shape[1], w.shape[1]

    def body(xb, d0b, d1b, wb, ob):
        ob[...] = _dis(d0b, d1b) * jnp.dot(
            xb[...], wb[...], preferred_element_type=jnp.float32)

    return pl.pallas_call(
        body,
        grid=(GRID,),
        in_specs=[pl.BlockSpec((BM, kdim), lambda i: (i, 0)),
                  _spec(16), _spec(16),
                  pl.BlockSpec((kdim, fout), lambda i: (0, 0))],
        out_specs=pl.BlockSpec((BM, fout), lambda i: (i, 0)),
        out_shape=jax.ShapeDtypeStruct((NPAD, fout), jnp.float32),
    )(xp, d0, d1, w)


def _layer(s4, gp, d0, d1, w, bprev):
    fin, fout = gp.shape[1], w.shape[1]

    def body(sa, sb, sc, sd, gb, d0b, d1b, bb, wb, ob):
        s = jnp.concatenate([sa[...], sb[...], sc[...], sd[...]], axis=1)
        dis = _dis(d0b, d1b)
        z = jnp.maximum(dis * (s + gb[...]) + bb[...], 0.0)
        ob[...] = dis * jnp.dot(z, wb[...],
                                preferred_element_type=jnp.float32)

    return pl.pallas_call(
        body,
        grid=(GRID,),
        in_specs=[_spec(32), _spec(32), _spec(32), _spec(32),
                  _spec(fin), _spec(16), _spec(16),
                  pl.BlockSpec((1, fin), lambda i: (0, 0)),
                  pl.BlockSpec((fin, fout), lambda i: (0, 0))],
        out_specs=pl.BlockSpec((BM, fout), lambda i: (i, 0)),
        out_shape=jax.ShapeDtypeStruct((NPAD, fout), jnp.float32),
    )(s4[0], s4[1], s4[2], s4[3], gp, d0, d1, bprev, w)


def _final(s0, s1, gp, d0, d1, bcp):
    def body(sa, sb, gb, d0b, d1b, bb, ob):
        logits = _dis(d0b, d1b) * (sa[...] + sb[...] + gb[...]) + bb[...]
        colmask = lax.broadcasted_iota(jnp.int32, (BM, 16), 1) < 10
        masked = jnp.where(colmask, logits, jnp.float32(-1e30))
        m = jnp.max(masked, axis=1, keepdims=True)
        ssum = jnp.sum(jnp.where(colmask, jnp.exp(logits - m), 0.0),
                       axis=1, keepdims=True)
        ob[...] = logits - m - jnp.log(ssum)

    return pl.pallas_call(
        body,
        grid=(GRID,),
        in_specs=[_spec(16), _spec(16), _spec(16), _spec(16), _spec(16),
                  pl.BlockSpec((1, 16), lambda i: (0, 0))],
        out_specs=_spec(16),
        out_shape=jax.ShapeDtypeStruct((NPAD, 16), jnp.float32),
    )(s0, s1, gp, d0, d1, bcp)


def kernel(x, edge_index, edge_attr, batch, W1, b1, W2, b2, Wc, bc):
    xp = jnp.pad(x, ((0, NPAD - N), (0, 0)))
    src = edge_index[0]
    dst2 = edge_index[1].reshape(NGRP, GB)
    wcp = jnp.pad(Wc, ((0, 0), (0, 16 - Wc.shape[1])))
    bcp = jnp.pad(bc, (0, 16 - bc.shape[0])).reshape(1, 16)
    b1r = b1.reshape(1, -1)
    b2r = b2.reshape(1, -1)
    ones = jnp.ones((NPAD, 16), jnp.float32)

    prop_f = _prop_fsplit()
    prop_e = _prop_esplit()

    sdeg = prop_e(ones, src, dst2)
    d0, d1 = sdeg[0], sdeg[1]
    g1 = _mm_scale(xp, d0, d1, W1)
    s1 = prop_f(g1.reshape(4 * NPAD, 32), src, dst2)
    g2 = _layer(s1, g1, d0, d1, W2, b1r)
    s2 = prop_f(g2.reshape(4 * NPAD, 32), src, dst2)
    g3 = _layer(s2, g2, d0, d1, wcp, b2r)
    s3 = prop_e(g3, src, dst2)
    out16 = _final(s3[0], s3[1], g3, d0, d1, bcp)
    return out16[:N, :10]


# speedup vs baseline: 5.7958x; 1.0415x over previous
import functools

import jax
import jax.numpy as jnp
from jax import lax
from jax.experimental import pallas as pl
from jax.experimental.pallas import tpu as pltpu
from jax.experimental.pallas import tpu_sc as plsc

N = 50000
E = 800000
NPAD = 50176
RPT = NPAD // 16
ZB = 224
GB = 128
NGRP = E // GB

_SC_PARAMS = dict(
    mesh=plsc.VectorSubcoreMesh(core_axis_name="c", subcore_axis_name="s"),
    compiler_params=pltpu.CompilerParams(use_tc_tiling_on_sc=False),
)


def _zero_zbuf(zbuf, cols):
    zeros16 = jnp.zeros((16,), jnp.float32)

    def body(r, carry):
        for k in range(cols // 16):
            zbuf[r, pl.ds(k * 16, 16)] = zeros16
        return carry

    lax.fori_loop(0, ZB, body, 0)


def _zero_acc(acc, zbuf, sid):
    for k in range(RPT // ZB):
        pltpu.sync_copy(zbuf, acc.at[pl.ds((sid * (RPT // ZB) + k) * ZB, ZB)])


def _prop_fsplit():
    @functools.partial(
        pl.kernel,
        out_type=[jax.ShapeDtypeStruct((NPAD, 32), jnp.float32)] * 4,
        scratch_types=[
            pltpu.VMEM_SHARED((NPAD, 32), jnp.float32),
            pltpu.VMEM((GB, 32), jnp.float32),
            pltpu.VMEM((ZB, 32), jnp.float32),
            pltpu.VMEM((GB,), jnp.int32),
            pltpu.VMEM((1, GB), jnp.int32),
        ],
        **_SC_PARAMS,
    )
    def prop(g0, g1, g2, g3, src_hbm, dst2_hbm, o0, o1, o2, o3,
             acc, stage, zbuf, gidx, dstv):
        cid = lax.axis_index("c")
        sid = lax.axis_index("s")
        _zero_zbuf(zbuf, 32)
        ngrp = jnp.where(sid < 10, 391, 390)
        lo = sid * 390 + jnp.minimum(sid, 10)
        tables = (g0, g1, g2, g3)
        outs = (o0, o1, o2, o3)

        for p in range(4):
            @pl.when(cid == p // 2)
            def _(p=p):
                _zero_acc(acc, zbuf, sid)
                plsc.subcore_barrier()

                def group_body(k, carry):
                    g = lo + k
                    pltpu.sync_copy(src_hbm.at[pl.ds(g * GB, GB)], gidx)
                    pltpu.sync_copy(dst2_hbm.at[pl.ds(g, 1)], dstv)
                    pltpu.sync_copy(tables[p].at[gidx], stage)
                    pltpu.sync_copy(stage, acc.at[dstv.at[0]], add=True)
                    return carry

                lax.fori_loop(0, ngrp, group_body, 0)
                plsc.subcore_barrier()
                pltpu.sync_copy(acc.at[pl.ds(sid * RPT, RPT)],
                                outs[p].at[pl.ds(sid * RPT, RPT)])
                plsc.subcore_barrier()

    return prop


def _prop_esplit():
    @functools.partial(
        pl.kernel,
        out_type=[jax.ShapeDtypeStruct((NPAD, 16), jnp.float32)] * 2,
        scratch_types=[
            pltpu.VMEM_SHARED((NPAD, 16), jnp.float32),
            pltpu.VMEM((GB, 16), jnp.float32),
            pltpu.VMEM((ZB, 16), jnp.float32),
            pltpu.VMEM((GB,), jnp.int32),
            pltpu.VMEM((1, GB), jnp.int32),
        ],
        **_SC_PARAMS,
    )
    def prop(g_hbm, src_hbm, dst2_hbm, o0, o1,
             acc, stage, zbuf, gidx, dstv):
        cid = lax.axis_index("c")
        sid = lax.axis_index("s")
        _zero_zbuf(zbuf, 16)
        half = NGRP // 2
        ngrp = jnp.where(sid < 5, 196, 195)
        lo = cid * half + sid * 195 + jnp.minimum(sid, 5)

        _zero_acc(acc, zbuf, sid)
        plsc.subcore_barrier()

        def group_body(k, carry):
            g = lo + k
            pltpu.sync_copy(src_hbm.at[pl.ds(g * GB, GB)], gidx)
            pltpu.sync_copy(dst2_hbm.at[pl.ds(g, 1)], dstv)
            pltpu.sync_copy(g_hbm.at[gidx], stage)
            pltpu.sync_copy(stage, acc.at[dstv.at[0]], add=True)
            return carry

        lax.fori_loop(0, ngrp, group_body, 0)
        plsc.subcore_barrier()
        for p in range(2):
            @pl.when(cid == p)
            def _(p=p):
                pltpu.sync_copy(acc.at[pl.ds(sid * RPT, RPT)],
                                (o0, o1)[p].at[pl.ds(sid * RPT, RPT)])

    return prop


def _prop_deg():
    @functools.partial(
        pl.kernel,
        out_type=[jax.ShapeDtypeStruct((NPAD, 16), jnp.float32)] * 2,
        scratch_types=[
            pltpu.VMEM_SHARED((NPAD, 16), jnp.float32),
            pltpu.VMEM((GB, 16), jnp.float32),
            pltpu.VMEM((ZB, 16), jnp.float32),
            pltpu.VMEM((1, GB), jnp.int32),
        ],
        **_SC_PARAMS,
    )
    def prop(dst2_hbm, o0, o1, acc, stage, zbuf, dstv):
        cid = lax.axis_index("c")
        sid = lax.axis_index("s")
        _zero_zbuf(zbuf, 16)
        ones16 = jnp.ones((16,), jnp.float32)

        def fill(r, carry):
            stage[r, pl.ds(0, 16)] = ones16
            return carry

        lax.fori_loop(0, GB, fill, 0)
        half = NGRP // 2
        ngrp = jnp.where(sid < 5, 196, 195)
        lo = cid * half + sid * 195 + jnp.minimum(sid, 5)

        _zero_acc(acc, zbuf, sid)
        plsc.subcore_barrier()

        def group_body(k, carry):
            pltpu.sync_copy(dst2_hbm.at[pl.ds(lo + k, 1)], dstv)
            pltpu.sync_copy(stage, acc.at[dstv.at[0]], add=True)
            return carry

        lax.fori_loop(0, ngrp, group_body, 0)
        plsc.subcore_barrier()
        for p in range(2):
            @pl.when(cid == p)
            def _(p=p):
                pltpu.sync_copy(acc.at[pl.ds(sid * RPT, RPT)],
                                (o0, o1)[p].at[pl.ds(sid * RPT, RPT)])

    return prop


BM = 448
GRID = NPAD // BM


def _dis(d0b, d1b):
    return lax.rsqrt(d0b[...][:, 0:1] + d1b[...][:, 0:1] + 1.0)


def _spec(cols):
    return pl.BlockSpec((BM, cols), lambda i: (i, 0))


def _split4(res, outs):
    for p, ob in enumerate(outs):
        ob[...] = res[:, 32 * p:32 * p + 32]


def _mm_scale(xp, d0, d1, w):
    kdim = xp.shape[1]

    def body(xb, d0b, d1b, wb, o0, o1, o2, o3):
        res = _dis(d0b, d1b) * jnp.dot(xb[...], wb[...],
                                       preferred_element_type=jnp.float32)
        _split4(res, (o0, o1, o2, o3))

    return pl.pallas_call(
        body,
        grid=(GRID,),
        in_specs=[pl.BlockSpec((BM, kdim), lambda i: (i, 0)),
                  _spec(16), _spec(16),
                  pl.BlockSpec((kdim, 128), lambda i: (0, 0))],
        out_specs=[_spec(32)] * 4,
        out_shape=[jax.ShapeDtypeStruct((NPAD, 32), jnp.float32)] * 4,
    )(xp, d0, d1, w)


def _layer(s4, g4, d0, d1, w, bprev):
    fout = w.shape[1]

    def body(sa, sb, sc, sd, ga, gb_, gc, gd, d0b, d1b, bb, wb, *outs):
        s = jnp.concatenate([sa[...], sb[...], sc[...], sd[...]], axis=1)
        g = jnp.concatenate([ga[...], gb_[...], gc[...], gd[...]], axis=1)
        dis = _dis(d0b, d1b)
        z = jnp.maximum(dis * (s + g) + bb[...], 0.0)
        res = dis * jnp.dot(z, wb[...], preferred_element_type=jnp.float32)
        if fout == 128:
            _split4(res, outs)
        else:
            outs[0][...] = res

    if fout == 128:
        out_specs = [_spec(32)] * 4
        out_shape = [jax.ShapeDtypeStruct((NPAD, 32), jnp.float32)] * 4
    else:
        out_specs = _spec(fout)
        out_shape = jax.ShapeDtypeStruct((NPAD, fout), jnp.float32)

    return pl.pallas_call(
        body,
        grid=(GRID,),
        in_specs=[_spec(32)] * 8 + [
            _spec(16), _spec(16),
            pl.BlockSpec((1, 128), lambda i: (0, 0)),
            pl.BlockSpec((128, fout), lambda i: (0, 0))],
        out_specs=out_specs,
        out_shape=out_shape,
    )(*s4, *g4, d0, d1, bprev, w)


def _final(s0, s1, gp, d0, d1, bcp):
    def body(sa, sb, gb, d0b, d1b, bb, ob):
        logits = _dis(d0b, d1b) * (sa[...] + sb[...] + gb[...]) + bb[...]
        colmask = lax.broadcasted_iota(jnp.int32, (BM, 16), 1) < 10
        masked = jnp.where(colmask, logits, jnp.float32(-1e30))
        m = jnp.max(masked, axis=1, keepdims=True)
        ssum = jnp.sum(jnp.where(colmask, jnp.exp(logits - m), 0.0),
                       axis=1, keepdims=True)
        ob[...] = logits - m - jnp.log(ssum)

    return pl.pallas_call(
        body,
        grid=(GRID,),
        in_specs=[_spec(16)] * 5 + [pl.BlockSpec((1, 16), lambda i: (0, 0))],
        out_specs=_spec(16),
        out_shape=jax.ShapeDtypeStruct((NPAD, 16), jnp.float32),
    )(s0, s1, gp, d0, d1, bcp)


def kernel(x, edge_index, edge_attr, batch, W1, b1, W2, b2, Wc, bc):
    xp = jnp.pad(x, ((0, NPAD - N), (0, 0)))
    src = edge_index[0]
    dst2 = edge_index[1].reshape(NGRP, GB)
    wcp = jnp.pad(Wc, ((0, 0), (0, 16 - Wc.shape[1])))
    bcp = jnp.pad(bc, (0, 16 - bc.shape[0])).reshape(1, 16)
    b1r = b1.reshape(1, -1)
    b2r = b2.reshape(1, -1)

    prop_f = _prop_fsplit()
    prop_e = _prop_esplit()
    prop_d = _prop_deg()

    d0, d1 = prop_d(dst2)
    g1 = _mm_scale(xp, d0, d1, W1)
    s1 = prop_f(*g1, src, dst2)
    g2 = _layer(s1, g1, d0, d1, W2, b1r)
    s2 = prop_f(*g2, src, dst2)
    g3 = _layer(s2, g2, d0, d1, wcp, b2r)
    s30, s31 = prop_e(g3, src, dst2)
    out16 = _final(s30, s31, g3, d0, d1, bcp)
    return out16[:N, :10]


# speedup vs baseline: 6.5961x; 1.1381x over previous
import functools

import jax
import jax.numpy as jnp
from jax import lax
from jax.experimental import pallas as pl
from jax.experimental.pallas import tpu as pltpu
from jax.experimental.pallas import tpu_sc as plsc

N = 50000
E = 800000
RPT = N // 16
ZB = 125
GB = 128
NGRP = E // GB

_SC_PARAMS = dict(
    mesh=plsc.VectorSubcoreMesh(core_axis_name="c", subcore_axis_name="s"),
    compiler_params=pltpu.CompilerParams(use_tc_tiling_on_sc=False),
)


def _zero_zbuf(zbuf, cols):
    zeros16 = jnp.zeros((16,), jnp.float32)

    def body(r, carry):
        for k in range(cols // 16):
            zbuf[r, pl.ds(k * 16, 16)] = zeros16
        return carry

    lax.fori_loop(0, ZB, body, 0)


def _zero_acc(acc, zbuf, sid):
    for k in range(RPT // ZB):
        pltpu.sync_copy(zbuf, acc.at[pl.ds((sid * (RPT // ZB) + k) * ZB, ZB)])


def _prop_fsplit():
    @functools.partial(
        pl.kernel,
        out_type=[jax.ShapeDtypeStruct((N, 32), jnp.float32)] * 4,
        scratch_types=[
            pltpu.VMEM_SHARED((N, 32), jnp.float32),
            pltpu.VMEM((GB, 32), jnp.float32),
            pltpu.VMEM((ZB, 32), jnp.float32),
            pltpu.VMEM((GB,), jnp.int32),
            pltpu.VMEM((1, GB), jnp.int32),
        ],
        **_SC_PARAMS,
    )
    def prop(g0, g1, g2, g3, src_hbm, dst2_hbm, o0, o1, o2, o3,
             acc, stage, zbuf, gidx, dstv):
        cid = lax.axis_index("c")
        sid = lax.axis_index("s")
        _zero_zbuf(zbuf, 32)
        ngrp = jnp.where(sid < 10, 391, 390)
        lo = sid * 390 + jnp.minimum(sid, 10)
        tables = (g0, g1, g2, g3)
        outs = (o0, o1, o2, o3)

        for p in range(4):
            @pl.when(cid == p // 2)
            def _(p=p):
                _zero_acc(acc, zbuf, sid)
                plsc.subcore_barrier()

                def group_body(k, carry):
                    g = lo + k
                    pltpu.sync_copy(src_hbm.at[pl.ds(g * GB, GB)], gidx)
                    pltpu.sync_copy(dst2_hbm.at[pl.ds(g, 1)], dstv)
                    pltpu.sync_copy(tables[p].at[gidx], stage)
                    pltpu.sync_copy(stage, acc.at[dstv.at[0]], add=True)
                    return carry

                lax.fori_loop(0, ngrp, group_body, 0)
                plsc.subcore_barrier()
                pltpu.sync_copy(acc.at[pl.ds(sid * RPT, RPT)],
                                outs[p].at[pl.ds(sid * RPT, RPT)])
                plsc.subcore_barrier()

    return prop


def _prop_esplit():
    @functools.partial(
        pl.kernel,
        out_type=[jax.ShapeDtypeStruct((N, 16), jnp.float32)] * 2,
        scratch_types=[
            pltpu.VMEM_SHARED((N, 16), jnp.float32),
            pltpu.VMEM((GB, 16), jnp.float32),
            pltpu.VMEM((ZB, 16), jnp.float32),
            pltpu.VMEM((GB,), jnp.int32),
            pltpu.VMEM((1, GB), jnp.int32),
        ],
        **_SC_PARAMS,
    )
    def prop(g_hbm, src_hbm, dst2_hbm, o0, o1,
             acc, stage, zbuf, gidx, dstv):
        cid = lax.axis_index("c")
        sid = lax.axis_index("s")
        _zero_zbuf(zbuf, 16)
        half = NGRP // 2
        ngrp = jnp.where(sid < 5, 196, 195)
        lo = cid * half + sid * 195 + jnp.minimum(sid, 5)

        _zero_acc(acc, zbuf, sid)
        plsc.subcore_barrier()

        def group_body(k, carry):
            g = lo + k
            pltpu.sync_copy(src_hbm.at[pl.ds(g * GB, GB)], gidx)
            pltpu.sync_copy(dst2_hbm.at[pl.ds(g, 1)], dstv)
            pltpu.sync_copy(g_hbm.at[gidx], stage)
            pltpu.sync_copy(stage, acc.at[dstv.at[0]], add=True)
            return carry

        lax.fori_loop(0, ngrp, group_body, 0)
        plsc.subcore_barrier()
        for p in range(2):
            @pl.when(cid == p)
            def _(p=p):
                pltpu.sync_copy(acc.at[pl.ds(sid * RPT, RPT)],
                                (o0, o1)[p].at[pl.ds(sid * RPT, RPT)])

    return prop


def _prop_deg():
    @functools.partial(
        pl.kernel,
        out_type=[jax.ShapeDtypeStruct((N, 16), jnp.float32)] * 2,
        scratch_types=[
            pltpu.VMEM_SHARED((N, 16), jnp.float32),
            pltpu.VMEM((GB, 16), jnp.float32),
            pltpu.VMEM((ZB, 16), jnp.float32),
            pltpu.VMEM((1, GB), jnp.int32),
        ],
        **_SC_PARAMS,
    )
    def prop(dst2_hbm, o0, o1, acc, stage, zbuf, dstv):
        cid = lax.axis_index("c")
        sid = lax.axis_index("s")
        _zero_zbuf(zbuf, 16)
        ones16 = jnp.ones((16,), jnp.float32)

        def fill(r, carry):
            stage[r, pl.ds(0, 16)] = ones16
            return carry

        lax.fori_loop(0, GB, fill, 0)
        half = NGRP // 2
        ngrp = jnp.where(sid < 5, 196, 195)
        lo = cid * half + sid * 195 + jnp.minimum(sid, 5)

        _zero_acc(acc, zbuf, sid)
        plsc.subcore_barrier()

        def group_body(k, carry):
            pltpu.sync_copy(dst2_hbm.at[pl.ds(lo + k, 1)], dstv)
            pltpu.sync_copy(stage, acc.at[dstv.at[0]], add=True)
            return carry

        lax.fori_loop(0, ngrp, group_body, 0)
        plsc.subcore_barrier()
        for p in range(2):
            @pl.when(cid == p)
            def _(p=p):
                pltpu.sync_copy(acc.at[pl.ds(sid * RPT, RPT)],
                                (o0, o1)[p].at[pl.ds(sid * RPT, RPT)])

    return prop


BM = 400
GRID = N // BM


def _dis(d0b, d1b):
    return lax.rsqrt(d0b[...][:, 0:1] + d1b[...][:, 0:1] + 1.0)


def _spec(cols):
    return pl.BlockSpec((BM, cols), lambda i: (i, 0))


def _split4(res, outs):
    for p, ob in enumerate(outs):
        ob[...] = res[:, 32 * p:32 * p + 32]


def _mm_scale(xp, d0, d1, w):
    kdim = xp.shape[1]

    def body(xb, d0b, d1b, wb, o0, o1, o2, o3):
        res = _dis(d0b, d1b) * jnp.dot(xb[...], wb[...],
                                       preferred_element_type=jnp.float32)
        _split4(res, (o0, o1, o2, o3))

    return pl.pallas_call(
        body,
        grid=(GRID,),
        in_specs=[pl.BlockSpec((BM, kdim), lambda i: (i, 0)),
                  _spec(16), _spec(16),
                  pl.BlockSpec((kdim, 128), lambda i: (0, 0))],
        out_specs=[_spec(32)] * 4,
        out_shape=[jax.ShapeDtypeStruct((N, 32), jnp.float32)] * 4,
    )(xp, d0, d1, w)


def _layer(s4, g4, d0, d1, w, bprev):
    fout = w.shape[1]

    def body(sa, sb, sc, sd, ga, gb_, gc, gd, d0b, d1b, bb, wb, *outs):
        s = jnp.concatenate([sa[...], sb[...], sc[...], sd[...]], axis=1)
        g = jnp.concatenate([ga[...], gb_[...], gc[...], gd[...]], axis=1)
        dis = _dis(d0b, d1b)
        z = jnp.maximum(dis * (s + g) + bb[...], 0.0)
        res = dis * jnp.dot(z, wb[...], preferred_element_type=jnp.float32)
        if fout == 128:
            _split4(res, outs)
        else:
            outs[0][...] = res

    if fout == 128:
        out_specs = [_spec(32)] * 4
        out_shape = [jax.ShapeDtypeStruct((N, 32), jnp.float32)] * 4
    else:
        out_specs = _spec(fout)
        out_shape = jax.ShapeDtypeStruct((N, fout), jnp.float32)

    return pl.pallas_call(
        body,
        grid=(GRID,),
        in_specs=[_spec(32)] * 8 + [
            _spec(16), _spec(16),
            pl.BlockSpec((1, 128), lambda i: (0, 0)),
            pl.BlockSpec((128, fout), lambda i: (0, 0))],
        out_specs=out_specs,
        out_shape=out_shape,
    )(*s4, *g4, d0, d1, bprev, w)


def _final(s0, s1, gp, d0, d1, bcp):
    def body(sa, sb, gb, d0b, d1b, bb, ob):
        logits = _dis(d0b, d1b) * (sa[...] + sb[...] + gb[...]) + bb[...]
        colmask = lax.broadcasted_iota(jnp.int32, (BM, 16), 1) < 10
        masked = jnp.where(colmask, logits, jnp.float32(-1e30))
        m = jnp.max(masked, axis=1, keepdims=True)
        ssum = jnp.sum(jnp.where(colmask, jnp.exp(logits - m), 0.0),
                       axis=1, keepdims=True)
        ob[...] = logits - m - jnp.log(ssum)

    return pl.pallas_call(
        body,
        grid=(GRID,),
        in_specs=[_spec(16)] * 5 + [pl.BlockSpec((1, 16), lambda i: (0, 0))],
        out_specs=_spec(16),
        out_shape=jax.ShapeDtypeStruct((N, 16), jnp.float32),
    )(s0, s1, gp, d0, d1, bcp)


def kernel(x, edge_index, edge_attr, batch, W1, b1, W2, b2, Wc, bc):
    src = edge_index[0]
    dst2 = edge_index[1].reshape(NGRP, GB)
    wcp = jnp.pad(Wc, ((0, 0), (0, 16 - Wc.shape[1])))
    bcp = jnp.pad(bc, (0, 16 - bc.shape[0])).reshape(1, 16)
    b1r = b1.reshape(1, -1)
    b2r = b2.reshape(1, -1)

    prop_f = _prop_fsplit()
    prop_e = _prop_esplit()
    prop_d = _prop_deg()

    d0, d1 = prop_d(dst2)
    g1 = _mm_scale(x, d0, d1, W1)
    s1 = prop_f(*g1, src, dst2)
    g2 = _layer(s1, g1, d0, d1, W2, b1r)
    s2 = prop_f(*g2, src, dst2)
    g3 = _layer(s2, g2, d0, d1, wcp, b2r)
    s30, s31 = prop_e(g3, src, dst2)
    out16 = _final(s30, s31, g3, d0, d1, bcp)
    return out16[:, :10]


# speedup vs baseline: 7.1288x; 1.0808x over previous
import functools

import jax
import jax.numpy as jnp
from jax import lax
from jax.experimental import pallas as pl
from jax.experimental.pallas import tpu as pltpu
from jax.experimental.pallas import tpu_sc as plsc

N = 50000
E = 800000
RPT = N // 16
ZB = 125
GB = 128
NGRP = E // GB

_SC_PARAMS = dict(
    mesh=plsc.VectorSubcoreMesh(core_axis_name="c", subcore_axis_name="s"),
    compiler_params=pltpu.CompilerParams(use_tc_tiling_on_sc=False),
)


def _zero_zbuf(zbuf, cols):
    zeros16 = jnp.zeros((16,), jnp.float32)

    def body(r, carry):
        for k in range(cols // 16):
            zbuf[r, pl.ds(k * 16, 16)] = zeros16
        return carry

    lax.fori_loop(0, ZB, body, 0)


def _zero_acc(acc, zbuf, sid):
    for k in range(RPT // ZB):
        pltpu.sync_copy(zbuf, acc.at[pl.ds((sid * (RPT // ZB) + k) * ZB, ZB)])


def _pipe_groups(table, src_hbm, dst2_hbm, acc, stages, gidxs, dstvs,
                 gsems, ssems, lo, ngrp):
    def load_idx(b, k):
        pltpu.sync_copy(src_hbm.at[pl.ds((lo + k) * GB, GB)], gidxs[b])
        pltpu.sync_copy(dst2_hbm.at[pl.ds(lo + k, 1)], dstvs[b])

    def fire_gather(b):
        pltpu.async_copy(table.at[gidxs[b]], stages[b], gsems[b])

    def wait_gather(b):
        pltpu.make_async_copy(table.at[gidxs[b]], stages[b], gsems[b]).wait()

    def fire_scatter(b):
        pltpu.async_copy(stages[b], acc.at[dstvs[b].at[0]], ssems[b],
                         add=True)

    def wait_scatter(b):
        pltpu.make_async_copy(stages[b], acc.at[dstvs[b].at[0]],
                              ssems[b]).wait()

    load_idx(0, 0)
    fire_gather(0)

    def body(j, carry):
        k1 = 2 * j + 1
        k2 = 2 * j + 2
        wait_gather(0)
        fire_scatter(0)

        @pl.when(k1 < ngrp)
        def _():
            @pl.when(j > 0)
            def _():
                wait_scatter(1)

            load_idx(1, k1)
            fire_gather(1)
            wait_gather(1)
            fire_scatter(1)

        @pl.when(k2 < ngrp)
        def _():
            wait_scatter(0)
            load_idx(0, k2)
            fire_gather(0)

        return carry

    lax.fori_loop(0, (ngrp + 1) // 2, body, 0)
    wait_scatter(0)
    wait_scatter(1)


def _prop_fsplit():
    @functools.partial(
        pl.kernel,
        out_type=[jax.ShapeDtypeStruct((N, 32), jnp.float32)] * 4,
        scratch_types=[
            pltpu.VMEM_SHARED((N, 32), jnp.float32),
            pltpu.VMEM((GB, 32), jnp.float32),
            pltpu.VMEM((GB, 32), jnp.float32),
            pltpu.VMEM((ZB, 32), jnp.float32),
            pltpu.VMEM((GB,), jnp.int32),
            pltpu.VMEM((GB,), jnp.int32),
            pltpu.VMEM((1, GB), jnp.int32),
            pltpu.VMEM((1, GB), jnp.int32),
            pltpu.SemaphoreType.DMA,
            pltpu.SemaphoreType.DMA,
            pltpu.SemaphoreType.DMA,
            pltpu.SemaphoreType.DMA,
        ],
        **_SC_PARAMS,
    )
    def prop(g0, g1, g2, g3, src_hbm, dst2_hbm, o0, o1, o2, o3,
             acc, st0, st1, zbuf, gi0, gi1, dv0, dv1, gs0, gs1, ss0, ss1):
        cid = lax.axis_index("c")
        sid = lax.axis_index("s")
        _zero_zbuf(zbuf, 32)
        ngrp = jnp.where(sid < 10, 391, 390)
        lo = sid * 390 + jnp.minimum(sid, 10)
        tables = (g0, g1, g2, g3)
        outs = (o0, o1, o2, o3)

        for p in range(4):
            @pl.when(cid == p // 2)
            def _(p=p):
                _zero_acc(acc, zbuf, sid)
                plsc.subcore_barrier()
                _pipe_groups(tables[p], src_hbm, dst2_hbm, acc,
                             (st0, st1), (gi0, gi1), (dv0, dv1),
                             (gs0, gs1), (ss0, ss1), lo, ngrp)
                plsc.subcore_barrier()
                pltpu.sync_copy(acc.at[pl.ds(sid * RPT, RPT)],
                                outs[p].at[pl.ds(sid * RPT, RPT)])
                plsc.subcore_barrier()

    return prop


def _prop_esplit():
    @functools.partial(
        pl.kernel,
        out_type=[jax.ShapeDtypeStruct((N, 16), jnp.float32)] * 2,
        scratch_types=[
            pltpu.VMEM_SHARED((N, 16), jnp.float32),
            pltpu.VMEM((GB, 16), jnp.float32),
            pltpu.VMEM((GB, 16), jnp.float32),
            pltpu.VMEM((ZB, 16), jnp.float32),
            pltpu.VMEM((GB,), jnp.int32),
            pltpu.VMEM((GB,), jnp.int32),
            pltpu.VMEM((1, GB), jnp.int32),
            pltpu.VMEM((1, GB), jnp.int32),
            pltpu.SemaphoreType.DMA,
            pltpu.SemaphoreType.DMA,
            pltpu.SemaphoreType.DMA,
            pltpu.SemaphoreType.DMA,
        ],
        **_SC_PARAMS,
    )
    def prop(g_hbm, src_hbm, dst2_hbm, o0, o1,
             acc, st0, st1, zbuf, gi0, gi1, dv0, dv1, gs0, gs1, ss0, ss1):
        cid = lax.axis_index("c")
        sid = lax.axis_index("s")
        _zero_zbuf(zbuf, 16)
        half = NGRP // 2
        ngrp = jnp.where(sid < 5, 196, 195)
        lo = cid * half + sid * 195 + jnp.minimum(sid, 5)

        _zero_acc(acc, zbuf, sid)
        plsc.subcore_barrier()
        _pipe_groups(g_hbm, src_hbm, dst2_hbm, acc,
                     (st0, st1), (gi0, gi1), (dv0, dv1),
                     (gs0, gs1), (ss0, ss1), lo, ngrp)
        plsc.subcore_barrier()
        for p in range(2):
            @pl.when(cid == p)
            def _(p=p):
                pltpu.sync_copy(acc.at[pl.ds(sid * RPT, RPT)],
                                (o0, o1)[p].at[pl.ds(sid * RPT, RPT)])

    return prop


def _prop_deg():
    @functools.partial(
        pl.kernel,
        out_type=[jax.ShapeDtypeStruct((N, 16), jnp.float32)] * 2,
        scratch_types=[
            pltpu.VMEM_SHARED((N, 16), jnp.float32),
            pltpu.VMEM((GB, 16), jnp.float32),
            pltpu.VMEM((ZB, 16), jnp.float32),
            pltpu.VMEM((1, GB), jnp.int32),
        ],
        **_SC_PARAMS,
    )
    def prop(dst2_hbm, o0, o1, acc, stage, zbuf, dstv):
        cid = lax.axis_index("c")
        sid = lax.axis_index("s")
        _zero_zbuf(zbuf, 16)
        ones16 = jnp.ones((16,), jnp.float32)

        def fill(r, carry):
            stage[r, pl.ds(0, 16)] = ones16
            return carry

        lax.fori_loop(0, GB, fill, 0)
        half = NGRP // 2
        ngrp = jnp.where(sid < 5, 196, 195)
        lo = cid * half + sid * 195 + jnp.minimum(sid, 5)

        _zero_acc(acc, zbuf, sid)
        plsc.subcore_barrier()

        def group_body(k, carry):
            pltpu.sync_copy(dst2_hbm.at[pl.ds(lo + k, 1)], dstv)
            pltpu.sync_copy(stage, acc.at[dstv.at[0]], add=True)
            return carry

        lax.fori_loop(0, ngrp, group_body, 0)
        plsc.subcore_barrier()
        for p in range(2):
            @pl.when(cid == p)
            def _(p=p):
                pltpu.sync_copy(acc.at[pl.ds(sid * RPT, RPT)],
                                (o0, o1)[p].at[pl.ds(sid * RPT, RPT)])

    return prop


BM = 400
GRID = N // BM


def _dis(d0b, d1b):
    return lax.rsqrt(d0b[...][:, 0:1] + d1b[...][:, 0:1] + 1.0)


def _spec(cols):
    return pl.BlockSpec((BM, cols), lambda i: (i, 0))


def _split4(res, outs):
    for p, ob in enumerate(outs):
        ob[...] = res[:, 32 * p:32 * p + 32]


def _mm_scale(xp, d0, d1, w):
    kdim = xp.shape[1]

    def body(xb, d0b, d1b, wb, o0, o1, o2, o3):
        res = _dis(d0b, d1b) * jnp.dot(xb[...], wb[...],
                                       preferred_element_type=jnp.float32)
        _split4(res, (o0, o1, o2, o3))

    return pl.pallas_call(
        body,
        grid=(GRID,),
        in_specs=[pl.BlockSpec((BM, kdim), lambda i: (i, 0)),
                  _spec(16), _spec(16),
                  pl.BlockSpec((kdim, 128), lambda i: (0, 0))],
        out_specs=[_spec(32)] * 4,
        out_shape=[jax.ShapeDtypeStruct((N, 32), jnp.float32)] * 4,
    )(xp, d0, d1, w)


def _layer(s4, g4, d0, d1, w, bprev):
    fout = w.shape[1]

    def body(sa, sb, sc, sd, ga, gb_, gc, gd, d0b, d1b, bb, wb, *outs):
        s = jnp.concatenate([sa[...], sb[...], sc[...], sd[...]], axis=1)
        g = jnp.concatenate([ga[...], gb_[...], gc[...], gd[...]], axis=1)
        dis = _dis(d0b, d1b)
        z = jnp.maximum(dis * (s + g) + bb[...], 0.0)
        res = dis * jnp.dot(z, wb[...], preferred_element_type=jnp.float32)
        if fout == 128:
            _split4(res, outs)
        else:
            outs[0][...] = res

    if fout == 128:
        out_specs = [_spec(32)] * 4
        out_shape = [jax.ShapeDtypeStruct((N, 32), jnp.float32)] * 4
    else:
        out_specs = _spec(fout)
        out_shape = jax.ShapeDtypeStruct((N, fout), jnp.float32)

    return pl.pallas_call(
        body,
        grid=(GRID,),
        in_specs=[_spec(32)] * 8 + [
            _spec(16), _spec(16),
            pl.BlockSpec((1, 128), lambda i: (0, 0)),
            pl.BlockSpec((128, fout), lambda i: (0, 0))],
        out_specs=out_specs,
        out_shape=out_shape,
    )(*s4, *g4, d0, d1, bprev, w)


def _final(s0, s1, gp, d0, d1, bcp):
    def body(sa, sb, gb, d0b, d1b, bb, ob):
        logits = _dis(d0b, d1b) * (sa[...] + sb[...] + gb[...]) + bb[...]
        colmask = lax.broadcasted_iota(jnp.int32, (BM, 16), 1) < 10
        masked = jnp.where(colmask, logits, jnp.float32(-1e30))
        m = jnp.max(masked, axis=1, keepdims=True)
        ssum = jnp.sum(jnp.where(colmask, jnp.exp(logits - m), 0.0),
                       axis=1, keepdims=True)
        ob[...] = logits - m - jnp.log(ssum)

    return pl.pallas_call(
        body,
        grid=(GRID,),
        in_specs=[_spec(16)] * 5 + [pl.BlockSpec((1, 16), lambda i: (0, 0))],
        out_specs=_spec(16),
        out_shape=jax.ShapeDtypeStruct((N, 16), jnp.float32),
    )(s0, s1, gp, d0, d1, bcp)


def kernel(x, edge_index, edge_attr, batch, W1, b1, W2, b2, Wc, bc):
    src = edge_index[0]
    dst2 = edge_index[1].reshape(NGRP, GB)
    wcp = jnp.pad(Wc, ((0, 0), (0, 16 - Wc.shape[1])))
    bcp = jnp.pad(bc, (0, 16 - bc.shape[0])).reshape(1, 16)
    b1r = b1.reshape(1, -1)
    b2r = b2.reshape(1, -1)

    prop_f = _prop_fsplit()
    prop_e = _prop_esplit()
    prop_d = _prop_deg()

    d0, d1 = prop_d(dst2)
    g1 = _mm_scale(x, d0, d1, W1)
    s1 = prop_f(*g1, src, dst2)
    g2 = _layer(s1, g1, d0, d1, W2, b1r)
    s2 = prop_f(*g2, src, dst2)
    g3 = _layer(s2, g2, d0, d1, wcp, b2r)
    s30, s31 = prop_e(g3, src, dst2)
    out16 = _final(s30, s31, g3, d0, d1, bcp)
    return out16[:, :10]


# speedup vs baseline: 10.2072x; 1.4318x over previous
import functools

import jax
import jax.numpy as jnp
from jax import lax
from jax.experimental import pallas as pl
from jax.experimental.pallas import tpu as pltpu
from jax.experimental.pallas import tpu_sc as plsc

N = 50000
E = 800000
RPT = N // 16
ZB = 125
GB = 128
NGRP = E // GB

_SC_PARAMS = dict(
    mesh=plsc.VectorSubcoreMesh(core_axis_name="c", subcore_axis_name="s"),
    compiler_params=pltpu.CompilerParams(use_tc_tiling_on_sc=False),
)


def _zero_zbuf(zbuf, cols):
    zeros16 = jnp.zeros((16,), jnp.float32)

    def body(r, carry):
        for k in range(cols // 16):
            zbuf[r, pl.ds(k * 16, 16)] = zeros16
        return carry

    lax.fori_loop(0, ZB, body, 0)


def _zero_acc(acc, zbuf, sid):
    for k in range(RPT // ZB):
        pltpu.sync_copy(zbuf, acc.at[pl.ds((sid * (RPT // ZB) + k) * ZB, ZB)])


SEC = 16
PADG = 6288


def _pipe_sections(table, srcp, dst2p, acc, stages, gidx, dva, dvb,
                   gsems, ssems, lo, ngrp, npairs):
    def fire_gather(s, b):
        pltpu.async_copy(table.at[gidx.at[pl.ds(b * GB, GB)]],
                         stages[s], gsems[s])

    def wait_gather(s, b):
        pltpu.make_async_copy(table.at[gidx.at[pl.ds(b * GB, GB)]],
                              stages[s], gsems[s]).wait()

    def fire_scatter(s, dv, b):
        pltpu.async_copy(stages[s], acc.at[dv.at[b]], ssems[s], add=True)

    def wait_scatter(s, dv, b):
        pltpu.make_async_copy(stages[s], acc.at[dv.at[b]], ssems[s]).wait()

    def section(tt, dv, dv_prev):
        kk0 = SEC * tt
        pltpu.sync_copy(srcp.at[pl.ds((lo + kk0) * GB, SEC * GB)], gidx)
        pltpu.sync_copy(dst2p.at[pl.ds(lo + kk0, SEC)], dv)

        @pl.when((kk0 - 2 >= 0) & (kk0 - 2 < ngrp))
        def _():
            wait_scatter(0, dv_prev, 14)

        fire_gather(0, 0)
        for b in range(SEC):
            s = b % 2
            wait_gather(s, b)

            @pl.when(kk0 + b < ngrp)
            def _(s=s, b=b, dv=dv):
                fire_scatter(s, dv, b)

            if b < SEC - 1:
                sp = (b + 1) % 2
                kprev = kk0 + b - 1

                @pl.when((kprev >= 0) & (kprev < ngrp))
                def _(sp=sp, b=b, dv=dv, dv_prev=dv_prev):
                    if b == 0:
                        wait_scatter(sp, dv_prev, 15)
                    else:
                        wait_scatter(sp, dv, b - 1)

                fire_gather(sp, b + 1)

    def body(j, carry):
        section(2 * j, dva, dvb)
        section(2 * j + 1, dvb, dva)
        return carry

    lax.fori_loop(0, npairs, body, 0)


def _prop_fsplit():
    @functools.partial(
        pl.kernel,
        out_type=[jax.ShapeDtypeStruct((N, 32), jnp.float32)] * 4,
        scratch_types=[
            pltpu.VMEM_SHARED((N, 32), jnp.float32),
            pltpu.VMEM((GB, 32), jnp.float32),
            pltpu.VMEM((GB, 32), jnp.float32),
            pltpu.VMEM((ZB, 32), jnp.float32),
            pltpu.VMEM((SEC * GB,), jnp.int32),
            pltpu.VMEM((SEC, GB), jnp.int32),
            pltpu.VMEM((SEC, GB), jnp.int32),
            pltpu.SemaphoreType.DMA,
            pltpu.SemaphoreType.DMA,
            pltpu.SemaphoreType.DMA,
            pltpu.SemaphoreType.DMA,
        ],
        **_SC_PARAMS,
    )
    def prop(g0, g1, g2, g3, src_hbm, dst2_hbm, o0, o1, o2, o3,
             acc, st0, st1, zbuf, gidx, dva, dvb, gs0, gs1, ss0, ss1):
        cid = lax.axis_index("c")
        sid = lax.axis_index("s")
        _zero_zbuf(zbuf, 32)
        ngrp = jnp.where(sid < 10, 391, 390)
        lo = sid * 390 + jnp.minimum(sid, 10)
        tables = (g0, g1, g2, g3)
        outs = (o0, o1, o2, o3)

        for p in range(4):
            @pl.when(cid == p // 2)
            def _(p=p):
                _zero_acc(acc, zbuf, sid)
                plsc.subcore_barrier()
                _pipe_sections(tables[p], src_hbm, dst2_hbm, acc,
                               (st0, st1), gidx, dva, dvb,
                               (gs0, gs1), (ss0, ss1), lo, ngrp, 13)
                plsc.subcore_barrier()
                pltpu.sync_copy(acc.at[pl.ds(sid * RPT, RPT)],
                                outs[p].at[pl.ds(sid * RPT, RPT)])
                plsc.subcore_barrier()

    return prop


def _prop_esplit():
    @functools.partial(
        pl.kernel,
        out_type=[jax.ShapeDtypeStruct((N, 16), jnp.float32)] * 2,
        scratch_types=[
            pltpu.VMEM_SHARED((N, 16), jnp.float32),
            pltpu.VMEM((GB, 16), jnp.float32),
            pltpu.VMEM((GB, 16), jnp.float32),
            pltpu.VMEM((ZB, 16), jnp.float32),
            pltpu.VMEM((SEC * GB,), jnp.int32),
            pltpu.VMEM((SEC, GB), jnp.int32),
            pltpu.VMEM((SEC, GB), jnp.int32),
            pltpu.SemaphoreType.DMA,
            pltpu.SemaphoreType.DMA,
            pltpu.SemaphoreType.DMA,
            pltpu.SemaphoreType.DMA,
        ],
        **_SC_PARAMS,
    )
    def prop(g_hbm, src_hbm, dst2_hbm, o0, o1,
             acc, st0, st1, zbuf, gidx, dva, dvb, gs0, gs1, ss0, ss1):
        cid = lax.axis_index("c")
        sid = lax.axis_index("s")
        _zero_zbuf(zbuf, 16)
        half = NGRP // 2
        ngrp = jnp.where(sid < 5, 196, 195)
        lo = cid * half + sid * 195 + jnp.minimum(sid, 5)

        _zero_acc(acc, zbuf, sid)
        plsc.subcore_barrier()
        _pipe_sections(g_hbm, src_hbm, dst2_hbm, acc,
                       (st0, st1), gidx, dva, dvb,
                       (gs0, gs1), (ss0, ss1), lo, ngrp, 7)
        plsc.subcore_barrier()
        for p in range(2):
            @pl.when(cid == p)
            def _(p=p):
                pltpu.sync_copy(acc.at[pl.ds(sid * RPT, RPT)],
                                (o0, o1)[p].at[pl.ds(sid * RPT, RPT)])

    return prop


def _prop_deg():
    @functools.partial(
        pl.kernel,
        out_type=[jax.ShapeDtypeStruct((N, 16), jnp.float32)] * 2,
        scratch_types=[
            pltpu.VMEM_SHARED((N, 16), jnp.float32),
            pltpu.VMEM((GB, 16), jnp.float32),
            pltpu.VMEM((ZB, 16), jnp.float32),
            pltpu.VMEM((1, GB), jnp.int32),
        ],
        **_SC_PARAMS,
    )
    def prop(dst2_hbm, o0, o1, acc, stage, zbuf, dstv):
        cid = lax.axis_index("c")
        sid = lax.axis_index("s")
        _zero_zbuf(zbuf, 16)
        ones16 = jnp.ones((16,), jnp.float32)

        def fill(r, carry):
            stage[r, pl.ds(0, 16)] = ones16
            return carry

        lax.fori_loop(0, GB, fill, 0)
        half = NGRP // 2
        ngrp = jnp.where(sid < 5, 196, 195)
        lo = cid * half + sid * 195 + jnp.minimum(sid, 5)

        _zero_acc(acc, zbuf, sid)
        plsc.subcore_barrier()

        def group_body(k, carry):
            pltpu.sync_copy(dst2_hbm.at[pl.ds(lo + k, 1)], dstv)
            pltpu.sync_copy(stage, acc.at[dstv.at[0]], add=True)
            return carry

        lax.fori_loop(0, ngrp, group_body, 0)
        plsc.subcore_barrier()
        for p in range(2):
            @pl.when(cid == p)
            def _(p=p):
                pltpu.sync_copy(acc.at[pl.ds(sid * RPT, RPT)],
                                (o0, o1)[p].at[pl.ds(sid * RPT, RPT)])

    return prop


BM = 400
GRID = N // BM


def _dis(d0b, d1b):
    return lax.rsqrt(d0b[...][:, 0:1] + d1b[...][:, 0:1] + 1.0)


def _spec(cols):
    return pl.BlockSpec((BM, cols), lambda i: (i, 0))


def _split4(res, outs):
    for p, ob in enumerate(outs):
        ob[...] = res[:, 32 * p:32 * p + 32]


def _mm_scale(xp, d0, d1, w):
    kdim = xp.shape[1]

    def body(xb, d0b, d1b, wb, o0, o1, o2, o3):
        res = _dis(d0b, d1b) * jnp.dot(xb[...], wb[...],
                                       preferred_element_type=jnp.float32)
        _split4(res, (o0, o1, o2, o3))

    return pl.pallas_call(
        body,
        grid=(GRID,),
        in_specs=[pl.BlockSpec((BM, kdim), lambda i: (i, 0)),
                  _spec(16), _spec(16),
                  pl.BlockSpec((kdim, 128), lambda i: (0, 0))],
        out_specs=[_spec(32)] * 4,
        out_shape=[jax.ShapeDtypeStruct((N, 32), jnp.float32)] * 4,
    )(xp, d0, d1, w)


def _layer(s4, g4, d0, d1, w, bprev):
    fout = w.shape[1]

    def body(sa, sb, sc, sd, ga, gb_, gc, gd, d0b, d1b, bb, wb, *outs):
        s = jnp.concatenate([sa[...], sb[...], sc[...], sd[...]], axis=1)
        g = jnp.concatenate([ga[...], gb_[...], gc[...], gd[...]], axis=1)
        dis = _dis(d0b, d1b)
        z = jnp.maximum(dis * (s + g) + bb[...], 0.0)
        res = dis * jnp.dot(z, wb[...], preferred_element_type=jnp.float32)
        if fout == 128:
            _split4(res, outs)
        else:
            outs[0][...] = res

    if fout == 128:
        out_specs = [_spec(32)] * 4
        out_shape = [jax.ShapeDtypeStruct((N, 32), jnp.float32)] * 4
    else:
        out_specs = _spec(fout)
        out_shape = jax.ShapeDtypeStruct((N, fout), jnp.float32)

    return pl.pallas_call(
        body,
        grid=(GRID,),
        in_specs=[_spec(32)] * 8 + [
            _spec(16), _spec(16),
            pl.BlockSpec((1, 128), lambda i: (0, 0)),
            pl.BlockSpec((128, fout), lambda i: (0, 0))],
        out_specs=out_specs,
        out_shape=out_shape,
    )(*s4, *g4, d0, d1, bprev, w)


def _final(s0, s1, gp, d0, d1, bcp):
    def body(sa, sb, gb, d0b, d1b, bb, ob):
        logits = _dis(d0b, d1b) * (sa[...] + sb[...] + gb[...]) + bb[...]
        colmask = lax.broadcasted_iota(jnp.int32, (BM, 16), 1) < 10
        masked = jnp.where(colmask, logits, jnp.float32(-1e30))
        m = jnp.max(masked, axis=1, keepdims=True)
        ssum = jnp.sum(jnp.where(colmask, jnp.exp(logits - m), 0.0),
                       axis=1, keepdims=True)
        ob[...] = logits - m - jnp.log(ssum)

    return pl.pallas_call(
        body,
        grid=(GRID,),
        in_specs=[_spec(16)] * 5 + [pl.BlockSpec((1, 16), lambda i: (0, 0))],
        out_specs=_spec(16),
        out_shape=jax.ShapeDtypeStruct((N, 16), jnp.float32),
    )(s0, s1, gp, d0, d1, bcp)


def kernel(x, edge_index, edge_attr, batch, W1, b1, W2, b2, Wc, bc):
    src = jnp.pad(edge_index[0], (0, PADG * GB - E))
    dst2 = jnp.pad(edge_index[1].reshape(NGRP, GB), ((0, PADG - NGRP), (0, 0)))
    wcp = jnp.pad(Wc, ((0, 0), (0, 16 - Wc.shape[1])))
    bcp = jnp.pad(bc, (0, 16 - bc.shape[0])).reshape(1, 16)
    b1r = b1.reshape(1, -1)
    b2r = b2.reshape(1, -1)

    prop_f = _prop_fsplit()
    prop_e = _prop_esplit()
    prop_d = _prop_deg()

    d0, d1 = prop_d(dst2)
    g1 = _mm_scale(x, d0, d1, W1)
    s1 = prop_f(*g1, src, dst2)
    g2 = _layer(s1, g1, d0, d1, W2, b1r)
    s2 = prop_f(*g2, src, dst2)
    g3 = _layer(s2, g2, d0, d1, wcp, b2r)
    s30, s31 = prop_e(g3, src, dst2)
    out16 = _final(s30, s31, g3, d0, d1, bcp)
    return out16[:, :10]
